# baseline (speedup 1.0000x reference)
import functools

import jax
import jax.numpy as jnp
from jax import lax
from jax.experimental import pallas as pl
from jax.experimental.pallas import tpu as pltpu
from jax.experimental.pallas import tpu_sc as plsc

D = 64
BATCH = 4096
SEQ = 200
N = BATCH * SEQ
VOCAB = 1000000
NC = 2
NS = 16
NW = NC * NS
BB = BATCH // NW
SCALE = 8.0

WIN = 128
NWIN_FULL = VOCAB // WIN
REM = VOCAB - NWIN_FULL * WIN
NWIN = NWIN_FULL + 1
KMAX = (NWIN + NW - 1) // NW


def _transpose_pairs(tin, tout, npairs):
    lanes = lax.iota(jnp.int32, 16)

    def pair(u, carry):
        j0 = jnp.full((16,), 2 * u, jnp.int32)
        j1 = j0 + 1
        for c in range(D // 16):
            v0 = plsc.load_gather(tin, [lanes + 16 * c, j0])
            tout[u, pl.ds(16 * c, 16)] = v0 * SCALE
            v1 = plsc.load_gather(tin, [lanes + 16 * c, j1])
            tout[u, pl.ds(D + 16 * c, 16)] = v1 * SCALE
        return carry

    lax.fori_loop(0, npairs, pair, 0, unroll=2)


def _body1(wt_hbm, wrem_hbm, t_hbm, tin0, tin1, tout0, tout1,
           isem0, isem1, osem0, osem1):
    wid = lax.axis_index("s") * NC + lax.axis_index("c")

    tins = (tin0, tin1)
    touts = (tout0, tout1)
    isems = (isem0, isem1)
    osems = (osem0, osem1)

    def win_of(k):
        return k * NW + wid

    def start_in(k, b):
        win = win_of(k)

        @pl.when(win < NWIN_FULL)
        def _():
            pltpu.async_copy(wt_hbm.at[:, pl.ds(win * WIN, WIN)], tins[b],
                             isems[b])

        @pl.when(win == NWIN_FULL)
        def _():
            pltpu.async_copy(wrem_hbm, tins[b], isems[b])

    def wait_in(k, b):
        win = win_of(k)

        @pl.when(win < NWIN_FULL)
        def _():
            pltpu.make_async_copy(wt_hbm.at[:, pl.ds(0, WIN)], tins[b],
                                  isems[b]).wait()

        @pl.when(win == NWIN_FULL)
        def _():
            pltpu.make_async_copy(wrem_hbm, tins[b], isems[b]).wait()

    def start_out(k, b):
        win = win_of(k)

        @pl.when(win < NWIN_FULL)
        def _():
            pltpu.async_copy(touts[b],
                             t_hbm.at[pl.ds(win * (WIN // 2), WIN // 2)],
                             osems[b])

        @pl.when(win == NWIN_FULL)
        def _():
            pltpu.async_copy(touts[b].at[pl.ds(0, REM // 2)],
                             t_hbm.at[pl.ds(NWIN_FULL * (WIN // 2), REM // 2)],
                             osems[b])

    def wait_out(k, b):
        win = win_of(k)

        @pl.when(win < NWIN_FULL)
        def _():
            pltpu.make_async_copy(touts[b], t_hbm.at[pl.ds(0, WIN // 2)],
                                  osems[b]).wait()

        @pl.when(win == NWIN_FULL)
        def _():
            pltpu.make_async_copy(touts[b].at[pl.ds(0, REM // 2)],
                                  t_hbm.at[pl.ds(0, REM // 2)],
                                  osems[b]).wait()

    @pl.when(win_of(0) < NWIN)
    def _():
        start_in(0, 0)

    @pl.when(win_of(1) < NWIN)
    def _():
        start_in(1, 1)

    def step(k, carry):
        for bb in range(2):
            @pl.when((lax.rem(k, 2) == bb) & (win_of(k) < NWIN))
            def _():
                wait_in(k, bb)

                @pl.when(k >= 2)
                def _():
                    wait_out(k - 2, bb)

                @pl.when(win_of(k) < NWIN_FULL)
                def _():
                    _transpose_pairs(tins[bb], touts[bb], WIN // 2)

                @pl.when(win_of(k) == NWIN_FULL)
                def _():
                    _transpose_pairs(tins[bb], touts[bb], REM // 2)
                start_out(k, bb)

                @pl.when(win_of(k + 2) < NWIN)
                def _():
                    start_in(k + 2, bb)

        return carry

    lax.fori_loop(0, KMAX, step, 0)

    for k in (KMAX - 2, KMAX - 1):
        @pl.when(win_of(k) < NWIN)
        def _():
            for bb in range(2):
                @pl.when(lax.rem(k, 2) == bb)
                def _():
                    wait_out(k, bb)

    @pl.when(win_of(KMAX - 1) >= NWIN)
    def _():
        for bb in range(2):
            @pl.when(lax.rem(KMAX - 3, 2) == bb)
            def _():
                wait_out(KMAX - 3, bb)


def _transpose_block(pvrow, g, sb):
    lanes = lax.iota(jnp.int32, 16)

    for bg in range(BB // 16):
        pbase = pvrow[pl.ds(16 * bg, 16)] * D
        rows = lanes + 16 * bg

        def feat(e, carry):
            v = plsc.load_gather(g, [rows, pbase + e])
            sb[e, pl.ds(16 * bg, 16)] = v
            return carry

        lax.fori_loop(0, D, feat, 0, unroll=4)


NG = SEQ // 8


def _body2(t_hbm, idxt_hbm, out_hbm, ibg0, ibg1, id20, id21, pv0, pv1,
           g0, g1, s0, s1, xsem0, xsem1, gsem0, gsem1, ssem0, ssem1):
    wid = lax.axis_index("s") * NC + lax.axis_index("c")
    b0 = wid * BB

    ibgs = (ibg0, ibg1)
    id2s = (id20, id21)
    pvs = (pv0, pv1)
    gbufs = (g0, g1)
    sbufs = (s0, s1)
    xsems = (xsem0, xsem1)
    gsems = (gsem0, gsem1)
    ssems = (ssem0, ssem1)

    def start_idxg(grp, gb):
        pltpu.async_copy(idxt_hbm.at[pl.ds(8 * grp, 8), pl.ds(b0, BB)],
                         ibgs[gb], xsems[gb])

    def wait_idxg(gb):
        pltpu.make_async_copy(idxt_hbm.at[pl.ds(0, 8), pl.ds(0, BB)],
                              ibgs[gb], xsems[gb]).wait()

    def start_gather(b):
        pltpu.async_copy(t_hbm.at[id2s[b]], gbufs[b], gsems[b])

    def wait_gather(b):
        pltpu.make_async_copy(t_hbm.at[id2s[b]], gbufs[b], gsems[b]).wait()

    def start_scatter(s, b):
        pltpu.async_copy(sbufs[b], out_hbm.at[s, :, pl.ds(b0, BB)], ssems[b])

    def wait_scatter(b):
        pltpu.make_async_copy(sbufs[b], out_hbm.at[0, :, pl.ds(0, BB)],
                              ssems[b]).wait()

    def build(gb, r, b):
        row = ibgs[gb].at[r]
        for k in range(BB // 16):
            v = row[pl.ds(16 * k, 16)]
            id2s[b][pl.ds(16 * k, 16)] = v >> 1
            pvs[b][pl.ds(16 * k, 16)] = v & 1

    def dispatch(expr, fn):
        for gb in range(2):
            @pl.when(lax.rem(expr, 2) == gb)
            def _():
                fn(gb)

    start_idxg(0, 0)
    wait_idxg(0)
    build(0, 0, 0)
    start_gather(0)

    def pair(t, carry):
        grp = t // 4
        sa = 2 * t
        sb_ = sa + 1

        @pl.when((lax.rem(sa, 8) == 0) & (grp + 1 < NG))
        def _():
            dispatch(grp + 1, lambda gb: start_idxg(grp + 1, gb))

        ra = lax.rem(sa, 8)
        dispatch(grp, lambda gb: build(gb, ra + 1, 1))
        start_gather(1)

        wait_gather(0)

        @pl.when(sa >= 2)
        def _():
            wait_scatter(0)

        _transpose_block(pvs[0], gbufs[0], sbufs[0])
        start_scatter(sa, 0)

        @pl.when(sb_ + 1 < SEQ)
        def _():
            rn = lax.rem(sb_ + 1, 8)

            @pl.when(rn == 0)
            def _():
                def f(gb):
                    wait_idxg(gb)
                    build(gb, 0, 0)
                dispatch(grp + 1, f)

            @pl.when(rn != 0)
            def _():
                dispatch(grp, lambda gb: build(gb, rn, 0))

            start_gather(0)

        wait_gather(1)

        @pl.when(sb_ >= 2)
        def _():
            wait_scatter(1)

        _transpose_block(pvs[1], gbufs[1], sbufs[1])
        start_scatter(sb_, 1)

        return carry

    lax.fori_loop(0, SEQ // 2, pair, 0)

    wait_scatter(0)
    wait_scatter(1)


def kernel(input_ids, W):
    wt = W.T
    wrem = jnp.pad(W[NWIN_FULL * WIN:].T, ((0, 0), (0, WIN - REM)))
    idxt = input_ids.T
    mesh = plsc.VectorSubcoreMesh(core_axis_name="c", subcore_axis_name="s")

    k1 = functools.partial(
        pl.kernel,
        mesh=mesh,
        compiler_params=pltpu.CompilerParams(needs_layout_passes=False),
        out_type=jax.ShapeDtypeStruct((VOCAB // 2, 2 * D), jnp.float32),
        scratch_types=[
            pltpu.VMEM((D, WIN), jnp.float32),
            pltpu.VMEM((D, WIN), jnp.float32),
            pltpu.VMEM((WIN // 2, 2 * D), jnp.float32),
            pltpu.VMEM((WIN // 2, 2 * D), jnp.float32),
            pltpu.SemaphoreType.DMA,
            pltpu.SemaphoreType.DMA,
            pltpu.SemaphoreType.DMA,
            pltpu.SemaphoreType.DMA,
        ],
    )(_body1)
    table = k1(wt, wrem)

    k2 = functools.partial(
        pl.kernel,
        mesh=mesh,
        compiler_params=pltpu.CompilerParams(needs_layout_passes=False),
        out_type=jax.ShapeDtypeStruct((SEQ, D, BATCH), jnp.float32),
        scratch_types=(
            [pltpu.VMEM((8, BB), jnp.int32) for _ in range(2)]
            + [pltpu.VMEM((BB,), jnp.int32) for _ in range(2)]
            + [pltpu.VMEM((BB,), jnp.int32) for _ in range(2)]
            + [pltpu.VMEM((BB, 2 * D), jnp.float32) for _ in range(2)]
            + [pltpu.VMEM((D, BB), jnp.float32) for _ in range(2)]
            + [pltpu.SemaphoreType.DMA for _ in range(6)]
        ),
    )(_body2)
    out = k2(table, idxt)
    return jnp.transpose(out, (2, 0, 1))


# speedup vs baseline: 2.3210x; 2.3210x over previous
import functools

import jax
import jax.numpy as jnp
from jax import lax
from jax.experimental import pallas as pl
from jax.experimental.pallas import tpu as pltpu
from jax.experimental.pallas import tpu_sc as plsc

D = 64
BATCH = 4096
SEQ = 200
N = BATCH * SEQ
VOCAB = 1000000
NC = 2
NS = 16
NW = NC * NS
BB = BATCH // NW
SCALE = 8.0

WIN = 128
NWIN_FULL = VOCAB // WIN
REM = VOCAB - NWIN_FULL * WIN
NWIN = NWIN_FULL + 1
KMAX = (NWIN + NW - 1) // NW


def _transpose_pairs(tin, tout, nu):
    lanes = lax.iota(jnp.int32, 16)

    for h in range(2):
        def utile(ut, carry):
            u0 = ut * 16

            def etile(et, carry2):
                e0 = et * 16
                for k in range(16):
                    rot = (lanes + k) & 15
                    v = plsc.load_gather(
                        tin, [e0 + lanes, 2 * (u0 + rot) + h])
                    plsc.store_scatter(
                        tout, [u0 + rot, h * D + e0 + lanes], v * SCALE)
                return carry2

            lax.fori_loop(0, D // 16, etile, 0)
            return carry

        lax.fori_loop(0, nu, utile, 0)


def _body1(wt_hbm, wrem_hbm, t_hbm, tin0, tin1, tout0, tout1,
           isem0, isem1, osem0, osem1):
    wid = lax.axis_index("s") * NC + lax.axis_index("c")

    tins = (tin0, tin1)
    touts = (tout0, tout1)
    isems = (isem0, isem1)
    osems = (osem0, osem1)

    def win_of(k):
        return k * NW + wid

    def start_in(k, b):
        win = win_of(k)

        @pl.when(win < NWIN_FULL)
        def _():
            pltpu.async_copy(wt_hbm.at[:, pl.ds(win * WIN, WIN)], tins[b],
                             isems[b])

        @pl.when(win == NWIN_FULL)
        def _():
            pltpu.async_copy(wrem_hbm, tins[b], isems[b])

    def wait_in(k, b):
        win = win_of(k)

        @pl.when(win < NWIN_FULL)
        def _():
            pltpu.make_async_copy(wt_hbm.at[:, pl.ds(0, WIN)], tins[b],
                                  isems[b]).wait()

        @pl.when(win == NWIN_FULL)
        def _():
            pltpu.make_async_copy(wrem_hbm, tins[b], isems[b]).wait()

    def start_out(k, b):
        win = win_of(k)

        @pl.when(win < NWIN_FULL)
        def _():
            pltpu.async_copy(touts[b],
                             t_hbm.at[pl.ds(win * (WIN // 2), WIN // 2)],
                             osems[b])

        @pl.when(win == NWIN_FULL)
        def _():
            pltpu.async_copy(touts[b].at[pl.ds(0, REM // 2)],
                             t_hbm.at[pl.ds(NWIN_FULL * (WIN // 2), REM // 2)],
                             osems[b])

    def wait_out(k, b):
        win = win_of(k)

        @pl.when(win < NWIN_FULL)
        def _():
            pltpu.make_async_copy(touts[b], t_hbm.at[pl.ds(0, WIN // 2)],
                                  osems[b]).wait()

        @pl.when(win == NWIN_FULL)
        def _():
            pltpu.make_async_copy(touts[b].at[pl.ds(0, REM // 2)],
                                  t_hbm.at[pl.ds(0, REM // 2)],
                                  osems[b]).wait()

    @pl.when(win_of(0) < NWIN)
    def _():
        start_in(0, 0)

    @pl.when(win_of(1) < NWIN)
    def _():
        start_in(1, 1)

    def step(k, carry):
        for bb in range(2):
            @pl.when((lax.rem(k, 2) == bb) & (win_of(k) < NWIN))
            def _():
                wait_in(k, bb)

                @pl.when(k >= 2)
                def _():
                    wait_out(k - 2, bb)

                @pl.when(win_of(k) < NWIN_FULL)
                def _():
                    _transpose_pairs(tins[bb], touts[bb], WIN // 32)

                @pl.when(win_of(k) == NWIN_FULL)
                def _():
                    _transpose_pairs(tins[bb], touts[bb], REM // 32)
                start_out(k, bb)

                @pl.when(win_of(k + 2) < NWIN)
                def _():
                    start_in(k + 2, bb)

        return carry

    lax.fori_loop(0, KMAX, step, 0)

    for k in (KMAX - 2, KMAX - 1):
        @pl.when(win_of(k) < NWIN)
        def _():
            for bb in range(2):
                @pl.when(lax.rem(k, 2) == bb)
                def _():
                    wait_out(k, bb)

    @pl.when(win_of(KMAX - 1) >= NWIN)
    def _():
        for bb in range(2):
            @pl.when(lax.rem(KMAX - 3, 2) == bb)
            def _():
                wait_out(KMAX - 3, bb)


def _transpose_block(pvrow, g, sb):
    lanes = lax.iota(jnp.int32, 16)

    for bg in range(BB // 16):
        b0 = 16 * bg
        pbase = pvrow[pl.ds(b0, 16)] * D

        def etile(et, carry):
            e0 = et * 16
            for k in range(16):
                rot = (lanes + k) & 15
                v = plsc.load_gather(g, [b0 + lanes, pbase + e0 + rot])
                plsc.store_scatter(sb, [e0 + rot, b0 + lanes], v)
            return carry

        lax.fori_loop(0, D // 16, etile, 0)


NG = SEQ // 8


def _body2(t_hbm, idxt_hbm, out_hbm, ibg0, ibg1, id20, id21, pv0, pv1,
           g0, g1, s0, s1, xsem0, xsem1, gsem0, gsem1, ssem0, ssem1):
    wid = lax.axis_index("s") * NC + lax.axis_index("c")
    b0 = wid * BB

    ibgs = (ibg0, ibg1)
    id2s = (id20, id21)
    pvs = (pv0, pv1)
    gbufs = (g0, g1)
    sbufs = (s0, s1)
    xsems = (xsem0, xsem1)
    gsems = (gsem0, gsem1)
    ssems = (ssem0, ssem1)

    def start_idxg(grp, gb):
        pltpu.async_copy(idxt_hbm.at[pl.ds(8 * grp, 8), pl.ds(b0, BB)],
                         ibgs[gb], xsems[gb])

    def wait_idxg(gb):
        pltpu.make_async_copy(idxt_hbm.at[pl.ds(0, 8), pl.ds(0, BB)],
                              ibgs[gb], xsems[gb]).wait()

    def start_gather(b):
        pltpu.async_copy(t_hbm.at[id2s[b]], gbufs[b], gsems[b])

    def wait_gather(b):
        pltpu.make_async_copy(t_hbm.at[id2s[b]], gbufs[b], gsems[b]).wait()

    def start_scatter(s, b):
        pltpu.async_copy(sbufs[b], out_hbm.at[s, :, pl.ds(b0, BB)], ssems[b])

    def wait_scatter(b):
        pltpu.make_async_copy(sbufs[b], out_hbm.at[0, :, pl.ds(0, BB)],
                              ssems[b]).wait()

    def build(gb, r, b):
        row = ibgs[gb].at[r]
        for k in range(BB // 16):
            v = row[pl.ds(16 * k, 16)]
            id2s[b][pl.ds(16 * k, 16)] = v >> 1
            pvs[b][pl.ds(16 * k, 16)] = v & 1

    def dispatch(expr, fn):
        for gb in range(2):
            @pl.when(lax.rem(expr, 2) == gb)
            def _():
                fn(gb)

    start_idxg(0, 0)
    wait_idxg(0)
    build(0, 0, 0)
    start_gather(0)

    def pair(t, carry):
        grp = t // 4
        sa = 2 * t
        sb_ = sa + 1

        @pl.when((lax.rem(sa, 8) == 0) & (grp + 1 < NG))
        def _():
            dispatch(grp + 1, lambda gb: start_idxg(grp + 1, gb))

        ra = lax.rem(sa, 8)
        dispatch(grp, lambda gb: build(gb, ra + 1, 1))
        start_gather(1)

        wait_gather(0)

        @pl.when(sa >= 2)
        def _():
            wait_scatter(0)

        _transpose_block(pvs[0], gbufs[0], sbufs[0])
        start_scatter(sa, 0)

        @pl.when(sb_ + 1 < SEQ)
        def _():
            rn = lax.rem(sb_ + 1, 8)

            @pl.when(rn == 0)
            def _():
                def f(gb):
                    wait_idxg(gb)
                    build(gb, 0, 0)
                dispatch(grp + 1, f)

            @pl.when(rn != 0)
            def _():
                dispatch(grp, lambda gb: build(gb, rn, 0))

            start_gather(0)

        wait_gather(1)

        @pl.when(sb_ >= 2)
        def _():
            wait_scatter(1)

        _transpose_block(pvs[1], gbufs[1], sbufs[1])
        start_scatter(sb_, 1)

        return carry

    lax.fori_loop(0, SEQ // 2, pair, 0)

    wait_scatter(0)
    wait_scatter(1)


def kernel(input_ids, W):
    wt = W.T
    wrem = jnp.pad(W[NWIN_FULL * WIN:].T, ((0, 0), (0, WIN - REM)))
    idxt = input_ids.T
    mesh = plsc.VectorSubcoreMesh(core_axis_name="c", subcore_axis_name="s")

    k1 = functools.partial(
        pl.kernel,
        mesh=mesh,
        compiler_params=pltpu.CompilerParams(needs_layout_passes=False),
        out_type=jax.ShapeDtypeStruct((VOCAB // 2, 2 * D), jnp.float32),
        scratch_types=[
            pltpu.VMEM((D, WIN), jnp.float32),
            pltpu.VMEM((D, WIN), jnp.float32),
            pltpu.VMEM((WIN // 2, 2 * D), jnp.float32),
            pltpu.VMEM((WIN // 2, 2 * D), jnp.float32),
            pltpu.SemaphoreType.DMA,
            pltpu.SemaphoreType.DMA,
            pltpu.SemaphoreType.DMA,
            pltpu.SemaphoreType.DMA,
        ],
    )(_body1)
    table = k1(wt, wrem)

    k2 = functools.partial(
        pl.kernel,
        mesh=mesh,
        compiler_params=pltpu.CompilerParams(needs_layout_passes=False),
        out_type=jax.ShapeDtypeStruct((SEQ, D, BATCH), jnp.float32),
        scratch_types=(
            [pltpu.VMEM((8, BB), jnp.int32) for _ in range(2)]
            + [pltpu.VMEM((BB,), jnp.int32) for _ in range(2)]
            + [pltpu.VMEM((BB,), jnp.int32) for _ in range(2)]
            + [pltpu.VMEM((BB, 2 * D), jnp.float32) for _ in range(2)]
            + [pltpu.VMEM((D, BB), jnp.float32) for _ in range(2)]
            + [pltpu.SemaphoreType.DMA for _ in range(6)]
        ),
    )(_body2)
    out = k2(table, idxt)
    return jnp.transpose(out, (2, 0, 1))


# speedup vs baseline: 4.7280x; 2.0371x over previous
import functools

import jax
import jax.numpy as jnp
from jax import lax
from jax.experimental import pallas as pl
from jax.experimental.pallas import tpu as pltpu
from jax.experimental.pallas import tpu_sc as plsc

D = 64
BATCH = 4096
SEQ = 200
N = BATCH * SEQ
VOCAB = 1000000
NC = 2
NS = 16
NW = NC * NS
BB = BATCH // NW
SCALE = 8.0

WIN = 128
NWIN_FULL = VOCAB // WIN
REM = VOCAB - NWIN_FULL * WIN
NWIN = NWIN_FULL + 1
KMAX = (NWIN + NW - 1) // NW


def _transpose_pairs(tin, tout, nu):
    lanes = lax.iota(jnp.int32, 16)

    for h in range(2):
        def utile(ut, carry):
            u0 = ut * 16

            def etile(et, carry2):
                e0 = et * 16
                vs = []
                for k in range(16):
                    rot = (lanes + k) & 15
                    vs.append((rot, plsc.load_gather(
                        tin, [e0 + lanes, 2 * (u0 + rot) + h])))
                for rot, v in vs:
                    plsc.store_scatter(
                        tout, [u0 + rot, h * D + e0 + lanes], v * SCALE)
                return carry2

            lax.fori_loop(0, D // 16, etile, 0)
            return carry

        lax.fori_loop(0, nu, utile, 0)


def _body1(wt_hbm, wrem_hbm, t_hbm, tin0, tin1, tout0, tout1,
           isem0, isem1, osem0, osem1):
    wid = lax.axis_index("s") * NC + lax.axis_index("c")

    tins = (tin0, tin1)
    touts = (tout0, tout1)
    isems = (isem0, isem1)
    osems = (osem0, osem1)

    def win_of(k):
        return k * NW + wid

    def start_in(k, b):
        win = win_of(k)

        @pl.when(win < NWIN_FULL)
        def _():
            pltpu.async_copy(wt_hbm.at[:, pl.ds(win * WIN, WIN)], tins[b],
                             isems[b])

        @pl.when(win == NWIN_FULL)
        def _():
            pltpu.async_copy(wrem_hbm, tins[b], isems[b])

    def wait_in(k, b):
        win = win_of(k)

        @pl.when(win < NWIN_FULL)
        def _():
            pltpu.make_async_copy(wt_hbm.at[:, pl.ds(0, WIN)], tins[b],
                                  isems[b]).wait()

        @pl.when(win == NWIN_FULL)
        def _():
            pltpu.make_async_copy(wrem_hbm, tins[b], isems[b]).wait()

    def start_out(k, b):
        win = win_of(k)

        @pl.when(win < NWIN_FULL)
        def _():
            pltpu.async_copy(touts[b],
                             t_hbm.at[pl.ds(win * (WIN // 2), WIN // 2)],
                             osems[b])

        @pl.when(win == NWIN_FULL)
        def _():
            pltpu.async_copy(touts[b].at[pl.ds(0, REM // 2)],
                             t_hbm.at[pl.ds(NWIN_FULL * (WIN // 2), REM // 2)],
                             osems[b])

    def wait_out(k, b):
        win = win_of(k)

        @pl.when(win < NWIN_FULL)
        def _():
            pltpu.make_async_copy(touts[b], t_hbm.at[pl.ds(0, WIN // 2)],
                                  osems[b]).wait()

        @pl.when(win == NWIN_FULL)
        def _():
            pltpu.make_async_copy(touts[b].at[pl.ds(0, REM // 2)],
                                  t_hbm.at[pl.ds(0, REM // 2)],
                                  osems[b]).wait()

    @pl.when(win_of(0) < NWIN)
    def _():
        start_in(0, 0)

    @pl.when(win_of(1) < NWIN)
    def _():
        start_in(1, 1)

    def step(k, carry):
        for bb in range(2):
            @pl.when((lax.rem(k, 2) == bb) & (win_of(k) < NWIN))
            def _():
                wait_in(k, bb)

                @pl.when(k >= 2)
                def _():
                    wait_out(k - 2, bb)

                @pl.when(win_of(k) < NWIN_FULL)
                def _():
                    _transpose_pairs(tins[bb], touts[bb], WIN // 32)

                @pl.when(win_of(k) == NWIN_FULL)
                def _():
                    _transpose_pairs(tins[bb], touts[bb], REM // 32)
                start_out(k, bb)

                @pl.when(win_of(k + 2) < NWIN)
                def _():
                    start_in(k + 2, bb)

        return carry

    lax.fori_loop(0, KMAX, step, 0)

    for k in (KMAX - 2, KMAX - 1):
        @pl.when(win_of(k) < NWIN)
        def _():
            for bb in range(2):
                @pl.when(lax.rem(k, 2) == bb)
                def _():
                    wait_out(k, bb)

    @pl.when(win_of(KMAX - 1) >= NWIN)
    def _():
        for bb in range(2):
            @pl.when(lax.rem(KMAX - 3, 2) == bb)
            def _():
                wait_out(KMAX - 3, bb)


def _transpose_block(pvrow, g, sb):
    lanes = lax.iota(jnp.int32, 16)

    for bg in range(BB // 16):
        b0 = 16 * bg
        pbase = pvrow[pl.ds(b0, 16)] * D

        def etile(et, carry):
            e0 = et * 16
            vs = []
            for k in range(16):
                rot = (lanes + k) & 15
                vs.append((rot, plsc.load_gather(
                    g, [b0 + lanes, pbase + e0 + rot])))
            for rot, v in vs:
                plsc.store_scatter(sb, [e0 + rot, b0 + lanes], v)
            return carry

        lax.fori_loop(0, D // 16, etile, 0)


NG = SEQ // 8


def _body2(t_hbm, idxt_hbm, out_hbm, ibg0, ibg1, id20, id21, pv0, pv1,
           g0, g1, s0, s1, xsem0, xsem1, gsem0, gsem1, ssem0, ssem1):
    wid = lax.axis_index("s") * NC + lax.axis_index("c")
    b0 = wid * BB

    ibgs = (ibg0, ibg1)
    id2s = (id20, id21)
    pvs = (pv0, pv1)
    gbufs = (g0, g1)
    sbufs = (s0, s1)
    xsems = (xsem0, xsem1)
    gsems = (gsem0, gsem1)
    ssems = (ssem0, ssem1)

    def start_idxg(grp, gb):
        pltpu.async_copy(idxt_hbm.at[pl.ds(8 * grp, 8), pl.ds(b0, BB)],
                         ibgs[gb], xsems[gb])

    def wait_idxg(gb):
        pltpu.make_async_copy(idxt_hbm.at[pl.ds(0, 8), pl.ds(0, BB)],
                              ibgs[gb], xsems[gb]).wait()

    def start_gather(b):
        pltpu.async_copy(t_hbm.at[id2s[b]], gbufs[b], gsems[b])

    def wait_gather(b):
        pltpu.make_async_copy(t_hbm.at[id2s[b]], gbufs[b], gsems[b]).wait()

    def start_scatter(s, b):
        pltpu.async_copy(sbufs[b], out_hbm.at[s, :, pl.ds(b0, BB)], ssems[b])

    def wait_scatter(b):
        pltpu.make_async_copy(sbufs[b], out_hbm.at[0, :, pl.ds(0, BB)],
                              ssems[b]).wait()

    def build(gb, r, b):
        row = ibgs[gb].at[r]
        for k in range(BB // 16):
            v = row[pl.ds(16 * k, 16)]
            id2s[b][pl.ds(16 * k, 16)] = v >> 1
            pvs[b][pl.ds(16 * k, 16)] = v & 1

    def dispatch(expr, fn):
        for gb in range(2):
            @pl.when(lax.rem(expr, 2) == gb)
            def _():
                fn(gb)

    start_idxg(0, 0)
    wait_idxg(0)
    build(0, 0, 0)
    start_gather(0)

    def pair(t, carry):
        grp = t // 4
        sa = 2 * t
        sb_ = sa + 1

        @pl.when((lax.rem(sa, 8) == 0) & (grp + 1 < NG))
        def _():
            dispatch(grp + 1, lambda gb: start_idxg(grp + 1, gb))

        ra = lax.rem(sa, 8)
        dispatch(grp, lambda gb: build(gb, ra + 1, 1))
        start_gather(1)

        wait_gather(0)

        @pl.when(sa >= 2)
        def _():
            wait_scatter(0)

        _transpose_block(pvs[0], gbufs[0], sbufs[0])
        start_scatter(sa, 0)

        @pl.when(sb_ + 1 < SEQ)
        def _():
            rn = lax.rem(sb_ + 1, 8)

            @pl.when(rn == 0)
            def _():
                def f(gb):
                    wait_idxg(gb)
                    build(gb, 0, 0)
                dispatch(grp + 1, f)

            @pl.when(rn != 0)
            def _():
                dispatch(grp, lambda gb: build(gb, rn, 0))

            start_gather(0)

        wait_gather(1)

        @pl.when(sb_ >= 2)
        def _():
            wait_scatter(1)

        _transpose_block(pvs[1], gbufs[1], sbufs[1])
        start_scatter(sb_, 1)

        return carry

    lax.fori_loop(0, SEQ // 2, pair, 0)

    wait_scatter(0)
    wait_scatter(1)


def kernel(input_ids, W):
    wt = W.T
    wrem = jnp.pad(W[NWIN_FULL * WIN:].T, ((0, 0), (0, WIN - REM)))
    idxt = input_ids.T
    mesh = plsc.VectorSubcoreMesh(core_axis_name="c", subcore_axis_name="s")

    k1 = functools.partial(
        pl.kernel,
        mesh=mesh,
        compiler_params=pltpu.CompilerParams(needs_layout_passes=False),
        out_type=jax.ShapeDtypeStruct((VOCAB // 2, 2 * D), jnp.float32),
        scratch_types=[
            pltpu.VMEM((D, WIN), jnp.float32),
            pltpu.VMEM((D, WIN), jnp.float32),
            pltpu.VMEM((WIN // 2, 2 * D), jnp.float32),
            pltpu.VMEM((WIN // 2, 2 * D), jnp.float32),
            pltpu.SemaphoreType.DMA,
            pltpu.SemaphoreType.DMA,
            pltpu.SemaphoreType.DMA,
            pltpu.SemaphoreType.DMA,
        ],
    )(_body1)
    table = k1(wt, wrem)

    k2 = functools.partial(
        pl.kernel,
        mesh=mesh,
        compiler_params=pltpu.CompilerParams(needs_layout_passes=False),
        out_type=jax.ShapeDtypeStruct((SEQ, D, BATCH), jnp.float32),
        scratch_types=(
            [pltpu.VMEM((8, BB), jnp.int32) for _ in range(2)]
            + [pltpu.VMEM((BB,), jnp.int32) for _ in range(2)]
            + [pltpu.VMEM((BB,), jnp.int32) for _ in range(2)]
            + [pltpu.VMEM((BB, 2 * D), jnp.float32) for _ in range(2)]
            + [pltpu.VMEM((D, BB), jnp.float32) for _ in range(2)]
            + [pltpu.SemaphoreType.DMA for _ in range(6)]
        ),
    )(_body2)
    out = k2(table, idxt)
    return jnp.transpose(out, (2, 0, 1))


# speedup vs baseline: 5.4501x; 1.1527x over previous
import functools

import jax
import jax.numpy as jnp
from jax import lax
from jax.experimental import pallas as pl
from jax.experimental.pallas import tpu as pltpu
from jax.experimental.pallas import tpu_sc as plsc

D = 64
BATCH = 4096
SEQ = 200
N = BATCH * SEQ
VOCAB = 1000000
NC = 2
NS = 16
NW = NC * NS
BB = BATCH // NW
SCALE = 8.0

WIN = 128
NWIN_FULL = VOCAB // WIN
REM = VOCAB - NWIN_FULL * WIN
NWIN = NWIN_FULL + 1
KMAX = (NWIN + NW - 1) // NW


def _transpose_pairs(tin, tout, nu):
    lanes = lax.iota(jnp.int32, 16)

    for h in range(2):
        def utile(ut, carry):
            u0 = ut * 16

            def etile(et, carry2):
                e0 = et * 16
                vs = []
                for k in range(16):
                    rot = (lanes + k) & 15
                    vs.append((rot, plsc.load_gather(
                        tin, [e0 + lanes, 2 * (u0 + rot) + h])))
                for rot, v in vs:
                    plsc.store_scatter(
                        tout, [u0 + rot, h * D + e0 + lanes], v * SCALE)
                return carry2

            lax.fori_loop(0, D // 16, etile, 0, unroll=2)
            return carry

        lax.fori_loop(0, nu, utile, 0)


def _body1(wt_hbm, wrem_hbm, t_hbm, tin0, tin1, tout0, tout1,
           isem0, isem1, osem0, osem1):
    wid = lax.axis_index("s") * NC + lax.axis_index("c")

    tins = (tin0, tin1)
    touts = (tout0, tout1)
    isems = (isem0, isem1)
    osems = (osem0, osem1)

    def win_of(k):
        return k * NW + wid

    def start_in(k, b):
        win = win_of(k)

        @pl.when(win < NWIN_FULL)
        def _():
            pltpu.async_copy(wt_hbm.at[:, pl.ds(win * WIN, WIN)], tins[b],
                             isems[b])

        @pl.when(win == NWIN_FULL)
        def _():
            pltpu.async_copy(wrem_hbm, tins[b], isems[b])

    def wait_in(k, b):
        win = win_of(k)

        @pl.when(win < NWIN_FULL)
        def _():
            pltpu.make_async_copy(wt_hbm.at[:, pl.ds(0, WIN)], tins[b],
                                  isems[b]).wait()

        @pl.when(win == NWIN_FULL)
        def _():
            pltpu.make_async_copy(wrem_hbm, tins[b], isems[b]).wait()

    def start_out(k, b):
        win = win_of(k)

        @pl.when(win < NWIN_FULL)
        def _():
            pltpu.async_copy(touts[b],
                             t_hbm.at[pl.ds(win * (WIN // 2), WIN // 2)],
                             osems[b])

        @pl.when(win == NWIN_FULL)
        def _():
            pltpu.async_copy(touts[b].at[pl.ds(0, REM // 2)],
                             t_hbm.at[pl.ds(NWIN_FULL * (WIN // 2), REM // 2)],
                             osems[b])

    def wait_out(k, b):
        win = win_of(k)

        @pl.when(win < NWIN_FULL)
        def _():
            pltpu.make_async_copy(touts[b], t_hbm.at[pl.ds(0, WIN // 2)],
                                  osems[b]).wait()

        @pl.when(win == NWIN_FULL)
        def _():
            pltpu.make_async_copy(touts[b].at[pl.ds(0, REM // 2)],
                                  t_hbm.at[pl.ds(0, REM // 2)],
                                  osems[b]).wait()

    @pl.when(win_of(0) < NWIN)
    def _():
        start_in(0, 0)

    @pl.when(win_of(1) < NWIN)
    def _():
        start_in(1, 1)

    def step(k, carry):
        for bb in range(2):
            @pl.when((lax.rem(k, 2) == bb) & (win_of(k) < NWIN))
            def _():
                wait_in(k, bb)

                @pl.when(k >= 2)
                def _():
                    wait_out(k - 2, bb)

                @pl.when(win_of(k) < NWIN_FULL)
                def _():
                    _transpose_pairs(tins[bb], touts[bb], WIN // 32)

                @pl.when(win_of(k) == NWIN_FULL)
                def _():
                    _transpose_pairs(tins[bb], touts[bb], REM // 32)
                start_out(k, bb)

                @pl.when(win_of(k + 2) < NWIN)
                def _():
                    start_in(k + 2, bb)

        return carry

    lax.fori_loop(0, KMAX, step, 0)

    for k in (KMAX - 2, KMAX - 1):
        @pl.when(win_of(k) < NWIN)
        def _():
            for bb in range(2):
                @pl.when(lax.rem(k, 2) == bb)
                def _():
                    wait_out(k, bb)

    @pl.when(win_of(KMAX - 1) >= NWIN)
    def _():
        for bb in range(2):
            @pl.when(lax.rem(KMAX - 3, 2) == bb)
            def _():
                wait_out(KMAX - 3, bb)


def _transpose_block(pvrow, g, sb):
    lanes = lax.iota(jnp.int32, 16)

    for bg in range(BB // 16):
        b0 = 16 * bg
        pbase = pvrow[pl.ds(b0, 16)] * D

        def etile(et, carry):
            e0 = et * 16
            vs = []
            for k in range(16):
                rot = (lanes + k) & 15
                vs.append((rot, plsc.load_gather(
                    g, [b0 + lanes, pbase + e0 + rot])))
            for rot, v in vs:
                plsc.store_scatter(sb, [e0 + rot, b0 + lanes], v)
            return carry

        lax.fori_loop(0, D // 16, etile, 0, unroll=2)


NG = SEQ // 8


def _body2(t_hbm, idxt_hbm, out_hbm, ibg0, ibg1, id20, id21, pv0, pv1,
           g0, g1, s0, s1, xsem0, xsem1, gsem0, gsem1, ssem0, ssem1):
    wid = lax.axis_index("s") * NC + lax.axis_index("c")
    b0 = wid * BB

    ibgs = (ibg0, ibg1)
    id2s = (id20, id21)
    pvs = (pv0, pv1)
    gbufs = (g0, g1)
    sbufs = (s0, s1)
    xsems = (xsem0, xsem1)
    gsems = (gsem0, gsem1)
    ssems = (ssem0, ssem1)

    def start_idxg(grp, gb):
        pltpu.async_copy(idxt_hbm.at[pl.ds(8 * grp, 8), pl.ds(b0, BB)],
                         ibgs[gb], xsems[gb])

    def wait_idxg(gb):
        pltpu.make_async_copy(idxt_hbm.at[pl.ds(0, 8), pl.ds(0, BB)],
                              ibgs[gb], xsems[gb]).wait()

    def start_gather(b):
        pltpu.async_copy(t_hbm.at[id2s[b]], gbufs[b], gsems[b])

    def wait_gather(b):
        pltpu.make_async_copy(t_hbm.at[id2s[b]], gbufs[b], gsems[b]).wait()

    def start_scatter(s, b):
        pltpu.async_copy(sbufs[b], out_hbm.at[s, :, pl.ds(b0, BB)], ssems[b])

    def wait_scatter(b):
        pltpu.make_async_copy(sbufs[b], out_hbm.at[0, :, pl.ds(0, BB)],
                              ssems[b]).wait()

    def build(gb, r, b):
        row = ibgs[gb].at[r]
        for k in range(BB // 16):
            v = row[pl.ds(16 * k, 16)]
            id2s[b][pl.ds(16 * k, 16)] = v >> 1
            pvs[b][pl.ds(16 * k, 16)] = v & 1

    def dispatch(expr, fn):
        for gb in range(2):
            @pl.when(lax.rem(expr, 2) == gb)
            def _():
                fn(gb)

    start_idxg(0, 0)
    wait_idxg(0)
    build(0, 0, 0)
    start_gather(0)

    def pair(t, carry):
        grp = t // 4
        sa = 2 * t
        sb_ = sa + 1

        @pl.when((lax.rem(sa, 8) == 0) & (grp + 1 < NG))
        def _():
            dispatch(grp + 1, lambda gb: start_idxg(grp + 1, gb))

        ra = lax.rem(sa, 8)
        dispatch(grp, lambda gb: build(gb, ra + 1, 1))
        start_gather(1)

        wait_gather(0)

        @pl.when(sa >= 2)
        def _():
            wait_scatter(0)

        _transpose_block(pvs[0], gbufs[0], sbufs[0])
        start_scatter(sa, 0)

        @pl.when(sb_ + 1 < SEQ)
        def _():
            rn = lax.rem(sb_ + 1, 8)

            @pl.when(rn == 0)
            def _():
                def f(gb):
                    wait_idxg(gb)
                    build(gb, 0, 0)
                dispatch(grp + 1, f)

            @pl.when(rn != 0)
            def _():
                dispatch(grp, lambda gb: build(gb, rn, 0))

            start_gather(0)

        wait_gather(1)

        @pl.when(sb_ >= 2)
        def _():
            wait_scatter(1)

        _transpose_block(pvs[1], gbufs[1], sbufs[1])
        start_scatter(sb_, 1)

        return carry

    lax.fori_loop(0, SEQ // 2, pair, 0)

    wait_scatter(0)
    wait_scatter(1)


def kernel(input_ids, W):
    wt = W.T
    wrem = jnp.pad(W[NWIN_FULL * WIN:].T, ((0, 0), (0, WIN - REM)))
    idxt = input_ids.T
    mesh = plsc.VectorSubcoreMesh(core_axis_name="c", subcore_axis_name="s")

    k1 = functools.partial(
        pl.kernel,
        mesh=mesh,
        compiler_params=pltpu.CompilerParams(needs_layout_passes=False),
        out_type=jax.ShapeDtypeStruct((VOCAB // 2, 2 * D), jnp.float32),
        scratch_types=[
            pltpu.VMEM((D, WIN), jnp.float32),
            pltpu.VMEM((D, WIN), jnp.float32),
            pltpu.VMEM((WIN // 2, 2 * D), jnp.float32),
            pltpu.VMEM((WIN // 2, 2 * D), jnp.float32),
            pltpu.SemaphoreType.DMA,
            pltpu.SemaphoreType.DMA,
            pltpu.SemaphoreType.DMA,
            pltpu.SemaphoreType.DMA,
        ],
    )(_body1)
    table = k1(wt, wrem)

    k2 = functools.partial(
        pl.kernel,
        mesh=mesh,
        compiler_params=pltpu.CompilerParams(needs_layout_passes=False),
        out_type=jax.ShapeDtypeStruct((SEQ, D, BATCH), jnp.float32),
        scratch_types=(
            [pltpu.VMEM((8, BB), jnp.int32) for _ in range(2)]
            + [pltpu.VMEM((BB,), jnp.int32) for _ in range(2)]
            + [pltpu.VMEM((BB,), jnp.int32) for _ in range(2)]
            + [pltpu.VMEM((BB, 2 * D), jnp.float32) for _ in range(2)]
            + [pltpu.VMEM((D, BB), jnp.float32) for _ in range(2)]
            + [pltpu.SemaphoreType.DMA for _ in range(6)]
        ),
    )(_body2)
    out = k2(table, idxt)
    return jnp.transpose(out, (2, 0, 1))
